# baseline (speedup 1.0000x reference)
import functools

import jax
import jax.numpy as jnp
from jax import lax
from jax.experimental import pallas as pl
from jax.experimental.pallas import tpu as pltpu
from jax.experimental.pallas import tpu_sc as plsc

EMBED = 128
SROW = 56
RPC = 2
NB = 4
GD = 2
SD = 2


@functools.partial(jax.jit, static_argnums=(2, 3, 4))
def _sc_gather(table, idx3, nw, b, s):
    mesh = plsc.VectorSubcoreMesh(core_axis_name="c", subcore_axis_name="s")
    rows_per_w = b // nw
    k_per_w = rows_per_w // RPC
    cpad = RPC * SROW
    assert k_per_w % NB == 0 and k_per_w >= NB

    @functools.partial(
        pl.kernel,
        mesh=mesh,
        out_type=jax.ShapeDtypeStruct((b * SROW, EMBED), jnp.float32),
        scratch_types=[
            pltpu.VMEM((k_per_w, cpad), jnp.int32),
            pltpu.VMEM((NB, cpad, EMBED), jnp.float32),
            pltpu.SemaphoreType.DMA((NB,)),
            pltpu.SemaphoreType.DMA((NB,)),
        ],
    )
    def k(table_hbm, idx_hbm, out_hbm, idx_v, rows_v, gsem, ssem):
        nc = 2
        wid = lax.axis_index("s") * nc + lax.axis_index("c")
        out_base = wid * rows_per_w * SROW
        pltpu.sync_copy(idx_hbm.at[wid], idx_v)

        def gather(j, slot):
            return pltpu.make_async_copy(
                table_hbm.at[idx_v.at[j]], rows_v.at[slot], gsem.at[slot])

        def store(j, slot):
            return pltpu.make_async_copy(
                rows_v.at[slot],
                out_hbm.at[pl.ds(out_base + j * cpad, cpad)],
                ssem.at[slot])

        for slot in range(GD):
            gather(slot, slot).start()

        def outer(i, _):
            g = i * NB
            for bslot in range(NB):
                j = g + bslot
                nslot = (bslot + GD) % NB
                @pl.when(j + GD - NB >= 0)
                def _():
                    store(j + GD - NB, nslot).wait()

                @pl.when(j + GD < k_per_w)
                def _():
                    gather(j + GD, nslot).start()

                gather(j, bslot).wait()
                store(j, bslot).start()
            return 0

        lax.fori_loop(0, k_per_w // NB, outer, 0)

        for j in range(k_per_w - SD, k_per_w):
            store(j, j % NB).wait()

    return k(table, idx3)


def kernel(cleavage_indices, pos_embed):
    b, s = cleavage_indices.shape
    info = plsc.get_sparse_core_info()
    nw = info.num_cores * info.num_subcores
    rows_per_w = b // nw
    k_per_w = rows_per_w // RPC
    idx = cleavage_indices.astype(jnp.int32).reshape(nw, rows_per_w, s)
    idx = jnp.pad(idx, ((0, 0), (0, 0), (0, SROW - s)))
    idx = idx.reshape(nw, k_per_w, RPC * SROW)
    out = _sc_gather(pos_embed, idx, nw, b, s)
    return out.reshape(b, SROW, EMBED)[:, :s, :]
